# baseline (speedup 1.0000x reference)
import functools

import jax
import jax.numpy as jnp
from jax import lax
from jax.experimental import pallas as pl
from jax.experimental.pallas import tpu as pltpu
from jax.experimental.pallas import tpu_sc as plsc

N = 50000
E = 800000
F_IN = 32
HID = 64
HEADS = 4
G = 64
EMB = 128

NC = 2
NS = 16
NWK = NC * NS
L = 16

NP = 50176
RPT = NP // NS

EDGES = E + N
W = 512
NWIN = 52
EP = NWK * W * NWIN
EPT = EP // NWK
EPA = EP + W

NB = 1024
NBLK = NP // NB

_sc_params = pltpu.CompilerParams(use_tc_tiling_on_sc=False)


def _mesh():
    return plsc.VectorSubcoreMesh(
        core_axis_name="c", subcore_axis_name="s",
        num_cores=NC, num_subcores=NS)


def _zero_shared(z_h, acc, sid):
    pltpu.sync_copy(z_h.at[pl.ds(sid * RPT, RPT)],
                    acc.at[pl.ds(sid * RPT, RPT)])


def _dump_shared(acc, out, c, sid):
    @pl.when(c == 0)
    def _c0():
        pltpu.sync_copy(acc.at[pl.ds(sid * RPT, RPT)],
                        out.at[0, pl.ds(sid * RPT, RPT)])

    @pl.when(c == 1)
    def _c1():
        pltpu.sync_copy(acc.at[pl.ds(sid * RPT, RPT)],
                        out.at[1, pl.ds(sid * RPT, RPT)])


def _make_edge_kernel(gat):
    out_type = [jax.ShapeDtypeStruct((EPA, L), jnp.float32)]
    if gat:
        out_type.append(jax.ShapeDtypeStruct((NC, NP, L), jnp.float32))
    scratch = [
        pltpu.VMEM((W,), jnp.int32), pltpu.VMEM((W,), jnp.int32),
        pltpu.VMEM((W,), jnp.int32), pltpu.VMEM((W,), jnp.int32),
        pltpu.VMEM((W, L), jnp.float32), pltpu.VMEM((W, L), jnp.float32),
        pltpu.VMEM((W, L), jnp.float32), pltpu.VMEM((W, L), jnp.float32),
        pltpu.SemaphoreType.DMA, pltpu.SemaphoreType.DMA,
        pltpu.SemaphoreType.DMA, pltpu.SemaphoreType.DMA,
    ]
    if gat:
        scratch.append(pltpu.VMEM_SHARED((NP, L), jnp.float32))

    @functools.partial(pl.kernel, out_type=tuple(out_type), mesh=_mesh(),
                       scratch_types=scratch, compiler_params=_sc_params)
    def edge_kernel(s_tab, d_tab, src_h, dst_h, z_h, *rest):
        it = iter(rest)
        ex_o = next(it)
        den_o = next(it) if gat else None
        srcb = [next(it), next(it)]
        dstb = [next(it), next(it)]
        sr = [next(it), next(it)]
        dr = [next(it), next(it)]
        sems = [next(it), next(it)], [next(it), next(it)]
        den_acc = next(it) if gat else None

        c = lax.axis_index("c")
        sid = lax.axis_index("s")
        wkr = sid * NC + c
        if gat:
            _zero_shared(z_h, den_acc, sid)
        plsc.subcore_barrier()

        def issue(b, w):
            base = wkr * EPT + w * W
            pltpu.sync_copy(src_h.at[pl.ds(base, W)], srcb[b])
            pltpu.sync_copy(dst_h.at[pl.ds(base, W)], dstb[b])
            g1 = pltpu.async_copy(s_tab.at[srcb[b]], sr[b], sems[0][b])
            g2 = pltpu.async_copy(d_tab.at[dstb[b]], dr[b], sems[1][b])
            return (g1, g2)

        def process(b, w, descs):
            base = wkr * EPT + w * W
            for d in descs:
                d.wait()

            def ebody(j, cc):
                if gat:
                    e = sr[b][j] + dr[b][j]
                    e = jnp.where(e >= 0.0, e, 0.2 * e)
                    sr[b][j] = jnp.exp(e)
                else:
                    sr[b][j] = sr[b][j] * dr[b][j]
                return cc

            lax.fori_loop(0, W, ebody, None, unroll=4)
            pltpu.sync_copy(sr[b], ex_o.at[pl.ds(base, W)])
            if gat:
                pltpu.sync_copy(sr[b], den_acc.at[dstb[b]], add=True)

        def pair(t, carry):
            w0 = 2 * t
            d0 = issue(0, w0)
            d1 = issue(1, w0 + 1)
            process(0, w0, d0)
            process(1, w0 + 1, d1)
            return carry

        lax.fori_loop(0, NWIN // 2, pair, None)
        if gat:
            plsc.subcore_barrier()
            _dump_shared(den_acc, den_o, c, sid)

    return edge_kernel


def _make_chunk_kernel(h):
    @functools.partial(
        pl.kernel,
        out_type=jax.ShapeDtypeStruct((NC, NP, L), jnp.float32),
        mesh=_mesh(),
        scratch_types=[
            pltpu.VMEM((W,), jnp.int32), pltpu.VMEM((W,), jnp.int32),
            pltpu.VMEM((W,), jnp.int32), pltpu.VMEM((W,), jnp.int32),
            pltpu.VMEM((W, L), jnp.float32), pltpu.VMEM((W, L), jnp.float32),
            pltpu.VMEM((W, L), jnp.float32), pltpu.VMEM((W, L), jnp.float32),
            pltpu.VMEM_SHARED((NP, L), jnp.float32),
            pltpu.SemaphoreType.DMA, pltpu.SemaphoreType.DMA,
            pltpu.SemaphoreType.DMA, pltpu.SemaphoreType.DMA,
        ],
        compiler_params=_sc_params)
    def chunk_kernel(ta, exr, src_h, dst_h, z_h, out, *bufs):
        it = iter(bufs)
        srcb = [next(it), next(it)]
        dstb = [next(it), next(it)]
        exb = [next(it), next(it)]
        rows_a = [next(it), next(it)]
        acc_a = next(it)
        sem_e = [next(it), next(it)]
        sem_a = [next(it), next(it)]

        c = lax.axis_index("c")
        sid = lax.axis_index("s")
        wkr = sid * NC + c
        _zero_shared(z_h, acc_a, sid)
        plsc.subcore_barrier()

        def issue(b, w):
            base = wkr * EPT + w * W
            pltpu.sync_copy(src_h.at[pl.ds(base, W)], srcb[b])
            pltpu.sync_copy(dst_h.at[pl.ds(base, W)], dstb[b])
            g1 = pltpu.async_copy(exr.at[pl.ds(base, W)], exb[b], sem_e[b])
            g2 = pltpu.async_copy(ta.at[srcb[b]], rows_a[b], sem_a[b])
            return (g1, g2)

        def process(b, w, descs):
            for d in descs:
                d.wait()

            def ebody(j, cc):
                ev = jnp.broadcast_to(exb[b][j][h:h + 1], (L,))
                rows_a[b][j] = rows_a[b][j] * ev
                return cc

            lax.fori_loop(0, W, ebody, None, unroll=4)
            pltpu.sync_copy(rows_a[b], acc_a.at[dstb[b]], add=True)

        def pair(t, carry):
            w0 = 2 * t
            d0 = issue(0, w0)
            d1 = issue(1, w0 + 1)
            process(0, w0, d0)
            process(1, w0 + 1, d1)
            return carry

        lax.fori_loop(0, NWIN // 2, pair, None)
        plsc.subcore_barrier()
        _dump_shared(acc_a, out, c, sid)

    return chunk_kernel


@functools.cache
def _sc_kernels():
    return {
        "gat": _make_edge_kernel(gat=True),
        "gcn": _make_edge_kernel(gat=False),
        "chunk": [_make_chunk_kernel(hh) for hh in range(4)],
    }


def _prep1_body(xp, w1, as1, ad1, *outs):
    xw_refs = outs[:16]
    s_o, d_o = outs[16], outs[17]
    xw = jnp.dot(xp[...], w1[...], preferred_element_type=jnp.float32)
    for t in range(16):
        xw_refs[t][...] = xw[:, t * L:(t + 1) * L]
    z12 = jnp.zeros((NB, L - HEADS), jnp.float32)
    s_cols = []
    d_cols = []
    for hh in range(HEADS):
        blk = xw[:, hh * HID:(hh + 1) * HID]
        s_cols.append(jnp.sum(blk * as1[...][hh][None, :], axis=1,
                              keepdims=True))
        d_cols.append(jnp.sum(blk * ad1[...][hh][None, :], axis=1,
                              keepdims=True))
    s_o[...] = jnp.concatenate(s_cols + [z12], axis=1)
    d_o[...] = jnp.concatenate(d_cols + [z12], axis=1)


def _prep2_body(*args):
    accs = args[:16]
    den, b1r, w2, as2, ad2 = args[16:21]
    outs = args[21:]
    xw2_refs = outs[:4]
    s2_o, d2_o, dinv_o = outs[4], outs[5], outs[6]
    dn = den[0] + den[1]
    parts = []
    for t in range(16):
        numt = accs[t][0] + accs[t][1]
        hh = t // 4
        ht = numt / (dn[:, hh:hh + 1] + 1e-16) \
            + b1r[...][:, t * L:(t + 1) * L]
        parts.append(jnp.maximum(ht, 0.0))
    h1 = jnp.concatenate(parts, axis=1)
    xw2 = jnp.dot(h1, w2[...], preferred_element_type=jnp.float32)
    for t in range(4):
        xw2_refs[t][...] = xw2[:, t * L:(t + 1) * L]
    z15 = jnp.zeros((NB, L - 1), jnp.float32)
    s2_o[...] = jnp.concatenate(
        [jnp.sum(xw2 * as2[...], axis=1, keepdims=True), z15], axis=1)
    d2_o[...] = jnp.concatenate(
        [jnp.sum(xw2 * ad2[...], axis=1, keepdims=True), z15], axis=1)
    deg = dn[:, HEADS:HEADS + 1]
    dinv = jnp.where(deg > 0.0, lax.rsqrt(jnp.maximum(deg, 1e-30)), 0.0)
    dinv_o[...] = jnp.concatenate([dinv, z15], axis=1)


def _prep3_body(a0, a1, a2, a3, den2, b2r, w3, x3a, x3b):
    num = jnp.concatenate([a0[0] + a0[1], a1[0] + a1[1], a2[0] + a2[1],
                           a3[0] + a3[1]], axis=1)
    dn = den2[0][:, 0:1] + den2[1][:, 0:1]
    h2 = jnp.maximum(num / (dn + 1e-16) + b2r[...], 0.0)
    xw3 = jnp.dot(h2, w3[...], preferred_element_type=jnp.float32)
    x3a[...] = xw3[:, :L]
    x3b[...] = xw3[:, L:]


def _final_body(a3a, a3b, b3r, batr, a1w, a1b, a2w, a2b, a3w, a3b_, gew, geb,
                an_o, emb_o, gm_acc, cnt_acc):
    i = pl.program_id(0)

    @pl.when(i == 0)
    def _init():
        gm_acc[...] = jnp.zeros_like(gm_acc)
        cnt_acc[...] = jnp.zeros_like(cnt_acc)

    h3 = jnp.concatenate([a3a[0] + a3a[1], a3b[0] + a3b[1]], axis=1)
    h3 = jnp.maximum(h3 + b3r[...], 0.0)
    bt = batr[...]
    gi = lax.broadcasted_iota(jnp.int32, (G, NB), 0).astype(jnp.float32)
    mask = (bt == gi).astype(jnp.float32)
    gm_acc[...] = gm_acc[...] + jnp.dot(mask, h3,
                                        preferred_element_type=jnp.float32)
    cnt_acc[...] = cnt_acc[...] + jnp.sum(mask, axis=1, keepdims=True)

    @pl.when(i == NBLK - 1)
    def _fin():
        gm = gm_acc[...] / jnp.maximum(cnt_acc[...], 1.0)
        a = jnp.maximum(jnp.dot(gm, a1w[...]) + a1b[...], 0.0)
        a = jnp.maximum(jnp.dot(a, a2w[...]) + a2b[...], 0.0)
        an_o[...] = jax.nn.sigmoid(jnp.dot(a, a3w[...]) + a3b_[...])
        emb_o[...] = jnp.tanh(jnp.dot(gm, gew[...]) + geb[...])


def _full(shape):
    return pl.BlockSpec(shape, lambda i: tuple(0 for _ in shape))


def _nblk(cols):
    return pl.BlockSpec((NB, cols), lambda i: (i, 0))


def _accblk(cols):
    return pl.BlockSpec((NC, NB, cols), lambda i: (0, i, 0))


def _sds(shape):
    return jax.ShapeDtypeStruct(shape, jnp.float32)


def kernel(x, edge_index, batch, W1, a_src1, a_dst1, b1, W2, a_src2, a_dst2,
           b2, W3, b3, A1w, A1b, A2w, A2b, A3w, A3b, GEw, GEb):
    f32 = jnp.float32
    xp = jnp.pad(x, ((0, NP - N), (0, 0)))
    loop = jnp.arange(N, dtype=jnp.int32)
    padi = N + (jnp.arange(EPA - EDGES, dtype=jnp.int32) % (NP - N))
    src = jnp.concatenate([edge_index[0], loop, padi])
    dst = jnp.concatenate([edge_index[1], loop, padi])
    batr = jnp.pad(batch, (0, NP - N), constant_values=G).astype(f32)
    batr = batr.reshape(1, NP)
    b1r = b1.reshape(1, -1)
    b2r = b2.reshape(1, -1)
    b3r = b3.reshape(1, -1)
    zeros16 = jnp.zeros((NP, L), f32)

    p1 = pl.pallas_call(
        _prep1_body,
        grid=(NBLK,),
        in_specs=[_nblk(F_IN), _full((F_IN, HEADS * HID)),
                  _full((HEADS, HID)), _full((HEADS, HID))],
        out_specs=[_nblk(L)] * 18,
        out_shape=[_sds((NP, L))] * 18,
    )(xp, W1, a_src1, a_dst1)
    xw1 = p1[:16]
    s1, d1 = p1[16], p1[17]

    sck = _sc_kernels()
    ex1, den1 = sck["gat"](s1, d1, src, dst, zeros16)
    acc1 = [sck["chunk"][t // 4](xw1[t], ex1, src, dst, zeros16)
            for t in range(16)]

    p2 = pl.pallas_call(
        _prep2_body,
        grid=(NBLK,),
        in_specs=[_accblk(L)] * 17
        + [_full((1, HEADS * HID)), _full((HEADS * HID, HID)),
           _full((1, HID)), _full((1, HID))],
        out_specs=[_nblk(L)] * 7,
        out_shape=[_sds((NP, L))] * 7,
    )(*acc1, den1, b1r, W2, a_src2, a_dst2)
    xw2 = p2[:4]
    s2, d2, dinv = p2[4], p2[5], p2[6]

    ex2, den2 = sck["gat"](s2, d2, src, dst, zeros16)
    a2_00 = sck["chunk"][0](xw2[0], ex2, src, dst, zeros16)
    a2_01 = sck["chunk"][0](xw2[1], ex2, src, dst, zeros16)
    a2_10 = sck["chunk"][0](xw2[2], ex2, src, dst, zeros16)
    a2_11 = sck["chunk"][0](xw2[3], ex2, src, dst, zeros16)

    x3a, x3b = pl.pallas_call(
        _prep3_body,
        grid=(NBLK,),
        in_specs=[_accblk(L)] * 5 + [_full((1, HID)),
                                     _full((HID, HID // 2))],
        out_specs=[_nblk(L), _nblk(L)],
        out_shape=[_sds((NP, L)), _sds((NP, L))],
    )(a2_00, a2_01, a2_10, a2_11, den2, b2r, W3)

    (norm,) = sck["gcn"](dinv, dinv, src, dst, zeros16)
    a3a = sck["chunk"][0](x3a, norm, src, dst, zeros16)
    a3b = sck["chunk"][0](x3b, norm, src, dst, zeros16)

    anomaly, emb = pl.pallas_call(
        _final_body,
        grid=(NBLK,),
        in_specs=[_accblk(L), _accblk(L), _full((1, 32)),
                  pl.BlockSpec((1, NB), lambda i: (0, i)),
                  _full((32, 32)), _full((1, 32)), _full((32, 16)),
                  _full((1, 16)), _full((16, 1)), _full((1, 1)),
                  _full((32, EMB)), _full((1, EMB))],
        out_specs=[_full((G, 1)), _full((G, EMB))],
        out_shape=[_sds((G, 1)), _sds((G, EMB))],
        scratch_shapes=[pltpu.VMEM((G, 32), f32), pltpu.VMEM((G, 1), f32)],
    )(a3a, a3b, b3r, batr, A1w, A1b.reshape(1, -1), A2w, A2b.reshape(1, -1),
      A3w, A3b.reshape(1, -1), GEw, GEb.reshape(1, -1))
    return (anomaly, emb)


# speedup vs baseline: 1.0154x; 1.0154x over previous
import functools

import jax
import jax.numpy as jnp
from jax import lax
from jax.experimental import pallas as pl
from jax.experimental.pallas import tpu as pltpu
from jax.experimental.pallas import tpu_sc as plsc

N = 50000
E = 800000
F_IN = 32
HID = 64
HEADS = 4
G = 64
EMB = 128

NC = 2
NS = 16
NWK = NC * NS
L = 16

NP = 50176
RPT = NP // NS

EDGES = E + N
W = 512
NWIN = 52
EP = NWK * W * NWIN
EPT = EP // NWK
EPA = EP + 2 * W

NB = 1024
NBLK = NP // NB

_sc_params = pltpu.CompilerParams(use_tc_tiling_on_sc=False)


def _mesh():
    return plsc.VectorSubcoreMesh(
        core_axis_name="c", subcore_axis_name="s",
        num_cores=NC, num_subcores=NS)


def _zero_shared(z_h, acc, sid):
    pltpu.sync_copy(z_h.at[pl.ds(sid * RPT, RPT)],
                    acc.at[pl.ds(sid * RPT, RPT)])


def _dump_shared(acc, out, c, sid):
    @pl.when(c == 0)
    def _c0():
        pltpu.sync_copy(acc.at[pl.ds(sid * RPT, RPT)],
                        out.at[0, pl.ds(sid * RPT, RPT)])

    @pl.when(c == 1)
    def _c1():
        pltpu.sync_copy(acc.at[pl.ds(sid * RPT, RPT)],
                        out.at[1, pl.ds(sid * RPT, RPT)])


def _make_edge_kernel(gat):
    out_type = [jax.ShapeDtypeStruct((EPA, L), jnp.float32)]
    if gat:
        out_type.append(jax.ShapeDtypeStruct((NC, NP, L), jnp.float32))
    scratch = [
        pltpu.VMEM((W,), jnp.int32), pltpu.VMEM((W,), jnp.int32),
        pltpu.VMEM((W,), jnp.int32), pltpu.VMEM((W,), jnp.int32),
        pltpu.VMEM((W, L), jnp.float32), pltpu.VMEM((W, L), jnp.float32),
        pltpu.VMEM((W, L), jnp.float32), pltpu.VMEM((W, L), jnp.float32),
        pltpu.SemaphoreType.DMA, pltpu.SemaphoreType.DMA,
        pltpu.SemaphoreType.DMA, pltpu.SemaphoreType.DMA,
        pltpu.SemaphoreType.DMA, pltpu.SemaphoreType.DMA,
    ]
    if gat:
        scratch.append(pltpu.VMEM_SHARED((NP, L), jnp.float32))

    @functools.partial(pl.kernel, out_type=tuple(out_type), mesh=_mesh(),
                       scratch_types=scratch, compiler_params=_sc_params)
    def edge_kernel(s_tab, d_tab, src_h, dst_h, z_h, *rest):
        it = iter(rest)
        ex_o = next(it)
        den_o = next(it) if gat else None
        srcb = [next(it), next(it)]
        dstb = [next(it), next(it)]
        sr = [next(it), next(it)]
        dr = [next(it), next(it)]
        sems = [next(it), next(it)], [next(it), next(it)]
        sem_i = [next(it), next(it)]
        den_acc = next(it) if gat else None

        c = lax.axis_index("c")
        sid = lax.axis_index("s")
        wkr = sid * NC + c
        if gat:
            _zero_shared(z_h, den_acc, sid)
        plsc.subcore_barrier()

        def issue_idx(b, w):
            base = wkr * EPT + w * W
            i1 = pltpu.async_copy(src_h.at[pl.ds(base, W)], srcb[b],
                                  sem_i[b])
            i2 = pltpu.async_copy(dst_h.at[pl.ds(base, W)], dstb[b],
                                  sem_i[b])
            return (i1, i2)

        def issue(b, idx_descs):
            for d in idx_descs:
                d.wait()
            g1 = pltpu.async_copy(s_tab.at[srcb[b]], sr[b], sems[0][b])
            g2 = pltpu.async_copy(d_tab.at[dstb[b]], dr[b], sems[1][b])
            return (g1, g2)

        def process(b, w, descs):
            base = wkr * EPT + w * W
            for d in descs:
                d.wait()

            def ebody(j, cc):
                if gat:
                    e = sr[b][j] + dr[b][j]
                    e = jnp.where(e >= 0.0, e, 0.2 * e)
                    sr[b][j] = jnp.exp(e)
                else:
                    sr[b][j] = sr[b][j] * dr[b][j]
                return cc

            lax.fori_loop(0, W, ebody, None, unroll=4)
            pltpu.sync_copy(sr[b], ex_o.at[pl.ds(base, W)])
            if gat:
                pltpu.sync_copy(sr[b], den_acc.at[dstb[b]], add=True)

        def pair(t, carry):
            w0 = 2 * t
            i0 = issue_idx(0, w0)
            i1 = issue_idx(1, w0 + 1)
            d0 = issue(0, i0)
            d1 = issue(1, i1)
            process(0, w0, d0)
            process(1, w0 + 1, d1)
            return carry

        lax.fori_loop(0, NWIN // 2, pair, None)
        if gat:
            plsc.subcore_barrier()
            _dump_shared(den_acc, den_o, c, sid)

    return edge_kernel


def _make_chunk_kernel(h):
    @functools.partial(
        pl.kernel,
        out_type=jax.ShapeDtypeStruct((NC, NP, L), jnp.float32),
        mesh=_mesh(),
        scratch_types=[
            pltpu.VMEM((W,), jnp.int32), pltpu.VMEM((W,), jnp.int32),
            pltpu.VMEM((W,), jnp.int32), pltpu.VMEM((W,), jnp.int32),
            pltpu.VMEM((W, L), jnp.float32), pltpu.VMEM((W, L), jnp.float32),
            pltpu.VMEM((W, L), jnp.float32), pltpu.VMEM((W, L), jnp.float32),
            pltpu.VMEM_SHARED((NP, L), jnp.float32),
            pltpu.SemaphoreType.DMA, pltpu.SemaphoreType.DMA,
            pltpu.SemaphoreType.DMA, pltpu.SemaphoreType.DMA,
            pltpu.SemaphoreType.DMA, pltpu.SemaphoreType.DMA,
        ],
        compiler_params=_sc_params)
    def chunk_kernel(ta, exr, src_h, dst_h, z_h, out, *bufs):
        it = iter(bufs)
        srcb = [next(it), next(it)]
        dstb = [next(it), next(it)]
        exb = [next(it), next(it)]
        rows_a = [next(it), next(it)]
        acc_a = next(it)
        sem_e = [next(it), next(it)]
        sem_a = [next(it), next(it)]
        sem_i = [next(it), next(it)]

        c = lax.axis_index("c")
        sid = lax.axis_index("s")
        wkr = sid * NC + c
        _zero_shared(z_h, acc_a, sid)
        plsc.subcore_barrier()

        def issue_idx(b, w):
            base = wkr * EPT + w * W
            i1 = pltpu.async_copy(src_h.at[pl.ds(base, W)], srcb[b],
                                  sem_i[b])
            i2 = pltpu.async_copy(dst_h.at[pl.ds(base, W)], dstb[b],
                                  sem_i[b])
            return (i1, i2)

        def issue(b, w, idx_descs):
            base = wkr * EPT + w * W
            g1 = pltpu.async_copy(exr.at[pl.ds(base, W)], exb[b], sem_e[b])
            for d in idx_descs:
                d.wait()
            g2 = pltpu.async_copy(ta.at[srcb[b]], rows_a[b], sem_a[b])
            return (g1, g2)

        def process(b, descs):
            for d in descs:
                d.wait()

            def ebody(j, cc):
                ev = jnp.broadcast_to(exb[b][j][h:h + 1], (L,))
                rows_a[b][j] = rows_a[b][j] * ev
                return cc

            lax.fori_loop(0, W, ebody, None, unroll=4)
            pltpu.sync_copy(rows_a[b], acc_a.at[dstb[b]], add=True)

        def pair(t, carry):
            w0 = 2 * t
            i0 = issue_idx(0, w0)
            i1 = issue_idx(1, w0 + 1)
            d0 = issue(0, w0, i0)
            d1 = issue(1, w0 + 1, i1)
            process(0, d0)
            process(1, d1)
            return carry

        lax.fori_loop(0, NWIN // 2, pair, None)
        plsc.subcore_barrier()
        _dump_shared(acc_a, out, c, sid)

    return chunk_kernel


@functools.cache
def _sc_kernels():
    return {
        "gat": _make_edge_kernel(gat=True),
        "gcn": _make_edge_kernel(gat=False),
        "chunk": [_make_chunk_kernel(hh) for hh in range(4)],
    }


def _prep1_body(xp, w1, as1, ad1, *outs):
    xw_refs = outs[:16]
    s_o, d_o = outs[16], outs[17]
    xw = jnp.dot(xp[...], w1[...], preferred_element_type=jnp.float32)
    for t in range(16):
        xw_refs[t][...] = xw[:, t * L:(t + 1) * L]
    z12 = jnp.zeros((NB, L - HEADS), jnp.float32)
    s_cols = []
    d_cols = []
    for hh in range(HEADS):
        blk = xw[:, hh * HID:(hh + 1) * HID]
        s_cols.append(jnp.sum(blk * as1[...][hh][None, :], axis=1,
                              keepdims=True))
        d_cols.append(jnp.sum(blk * ad1[...][hh][None, :], axis=1,
                              keepdims=True))
    s_o[...] = jnp.concatenate(s_cols + [z12], axis=1)
    d_o[...] = jnp.concatenate(d_cols + [z12], axis=1)


def _prep2_body(*args):
    accs = args[:16]
    den, b1r, w2, as2, ad2 = args[16:21]
    outs = args[21:]
    xw2_refs = outs[:4]
    s2_o, d2_o, dinv_o = outs[4], outs[5], outs[6]
    dn = den[0] + den[1]
    parts = []
    for t in range(16):
        numt = accs[t][0] + accs[t][1]
        hh = t // 4
        ht = numt / (dn[:, hh:hh + 1] + 1e-16) \
            + b1r[...][:, t * L:(t + 1) * L]
        parts.append(jnp.maximum(ht, 0.0))
    h1 = jnp.concatenate(parts, axis=1)
    xw2 = jnp.dot(h1, w2[...], preferred_element_type=jnp.float32)
    for t in range(4):
        xw2_refs[t][...] = xw2[:, t * L:(t + 1) * L]
    z15 = jnp.zeros((NB, L - 1), jnp.float32)
    s2_o[...] = jnp.concatenate(
        [jnp.sum(xw2 * as2[...], axis=1, keepdims=True), z15], axis=1)
    d2_o[...] = jnp.concatenate(
        [jnp.sum(xw2 * ad2[...], axis=1, keepdims=True), z15], axis=1)
    deg = dn[:, HEADS:HEADS + 1]
    dinv = jnp.where(deg > 0.0, lax.rsqrt(jnp.maximum(deg, 1e-30)), 0.0)
    dinv_o[...] = jnp.concatenate([dinv, z15], axis=1)


def _prep3_body(a0, a1, a2, a3, den2, b2r, w3, x3a, x3b):
    num = jnp.concatenate([a0[0] + a0[1], a1[0] + a1[1], a2[0] + a2[1],
                           a3[0] + a3[1]], axis=1)
    dn = den2[0][:, 0:1] + den2[1][:, 0:1]
    h2 = jnp.maximum(num / (dn + 1e-16) + b2r[...], 0.0)
    xw3 = jnp.dot(h2, w3[...], preferred_element_type=jnp.float32)
    x3a[...] = xw3[:, :L]
    x3b[...] = xw3[:, L:]


def _final_body(a3a, a3b, b3r, batr, a1w, a1b, a2w, a2b, a3w, a3b_, gew, geb,
                an_o, emb_o, gm_acc, cnt_acc):
    i = pl.program_id(0)

    @pl.when(i == 0)
    def _init():
        gm_acc[...] = jnp.zeros_like(gm_acc)
        cnt_acc[...] = jnp.zeros_like(cnt_acc)

    h3 = jnp.concatenate([a3a[0] + a3a[1], a3b[0] + a3b[1]], axis=1)
    h3 = jnp.maximum(h3 + b3r[...], 0.0)
    bt = batr[...]
    gi = lax.broadcasted_iota(jnp.int32, (G, NB), 0).astype(jnp.float32)
    mask = (bt == gi).astype(jnp.float32)
    gm_acc[...] = gm_acc[...] + jnp.dot(mask, h3,
                                        preferred_element_type=jnp.float32)
    cnt_acc[...] = cnt_acc[...] + jnp.sum(mask, axis=1, keepdims=True)

    @pl.when(i == NBLK - 1)
    def _fin():
        gm = gm_acc[...] / jnp.maximum(cnt_acc[...], 1.0)
        a = jnp.maximum(jnp.dot(gm, a1w[...]) + a1b[...], 0.0)
        a = jnp.maximum(jnp.dot(a, a2w[...]) + a2b[...], 0.0)
        an_o[...] = jax.nn.sigmoid(jnp.dot(a, a3w[...]) + a3b_[...])
        emb_o[...] = jnp.tanh(jnp.dot(gm, gew[...]) + geb[...])


def _full(shape):
    return pl.BlockSpec(shape, lambda i: tuple(0 for _ in shape))


def _nblk(cols):
    return pl.BlockSpec((NB, cols), lambda i: (i, 0))


def _accblk(cols):
    return pl.BlockSpec((NC, NB, cols), lambda i: (0, i, 0))


def _sds(shape):
    return jax.ShapeDtypeStruct(shape, jnp.float32)


def kernel(x, edge_index, batch, W1, a_src1, a_dst1, b1, W2, a_src2, a_dst2,
           b2, W3, b3, A1w, A1b, A2w, A2b, A3w, A3b, GEw, GEb):
    f32 = jnp.float32
    xp = jnp.pad(x, ((0, NP - N), (0, 0)))
    loop = jnp.arange(N, dtype=jnp.int32)
    padi = N + (jnp.arange(EPA - EDGES, dtype=jnp.int32) % (NP - N))
    src = jnp.concatenate([edge_index[0], loop, padi])
    dst = jnp.concatenate([edge_index[1], loop, padi])
    batr = jnp.pad(batch, (0, NP - N), constant_values=G).astype(f32)
    batr = batr.reshape(1, NP)
    b1r = b1.reshape(1, -1)
    b2r = b2.reshape(1, -1)
    b3r = b3.reshape(1, -1)
    zeros16 = jnp.zeros((NP, L), f32)

    p1 = pl.pallas_call(
        _prep1_body,
        grid=(NBLK,),
        in_specs=[_nblk(F_IN), _full((F_IN, HEADS * HID)),
                  _full((HEADS, HID)), _full((HEADS, HID))],
        out_specs=[_nblk(L)] * 18,
        out_shape=[_sds((NP, L))] * 18,
    )(xp, W1, a_src1, a_dst1)
    xw1 = p1[:16]
    s1, d1 = p1[16], p1[17]

    sck = _sc_kernels()
    ex1, den1 = sck["gat"](s1, d1, src, dst, zeros16)
    acc1 = [sck["chunk"][t // 4](xw1[t], ex1, src, dst, zeros16)
            for t in range(16)]

    p2 = pl.pallas_call(
        _prep2_body,
        grid=(NBLK,),
        in_specs=[_accblk(L)] * 17
        + [_full((1, HEADS * HID)), _full((HEADS * HID, HID)),
           _full((1, HID)), _full((1, HID))],
        out_specs=[_nblk(L)] * 7,
        out_shape=[_sds((NP, L))] * 7,
    )(*acc1, den1, b1r, W2, a_src2, a_dst2)
    xw2 = p2[:4]
    s2, d2, dinv = p2[4], p2[5], p2[6]

    ex2, den2 = sck["gat"](s2, d2, src, dst, zeros16)
    a2_00 = sck["chunk"][0](xw2[0], ex2, src, dst, zeros16)
    a2_01 = sck["chunk"][0](xw2[1], ex2, src, dst, zeros16)
    a2_10 = sck["chunk"][0](xw2[2], ex2, src, dst, zeros16)
    a2_11 = sck["chunk"][0](xw2[3], ex2, src, dst, zeros16)

    x3a, x3b = pl.pallas_call(
        _prep3_body,
        grid=(NBLK,),
        in_specs=[_accblk(L)] * 5 + [_full((1, HID)),
                                     _full((HID, HID // 2))],
        out_specs=[_nblk(L), _nblk(L)],
        out_shape=[_sds((NP, L)), _sds((NP, L))],
    )(a2_00, a2_01, a2_10, a2_11, den2, b2r, W3)

    (norm,) = sck["gcn"](dinv, dinv, src, dst, zeros16)
    a3a = sck["chunk"][0](x3a, norm, src, dst, zeros16)
    a3b = sck["chunk"][0](x3b, norm, src, dst, zeros16)

    anomaly, emb = pl.pallas_call(
        _final_body,
        grid=(NBLK,),
        in_specs=[_accblk(L), _accblk(L), _full((1, 32)),
                  pl.BlockSpec((1, NB), lambda i: (0, i)),
                  _full((32, 32)), _full((1, 32)), _full((32, 16)),
                  _full((1, 16)), _full((16, 1)), _full((1, 1)),
                  _full((32, EMB)), _full((1, EMB))],
        out_specs=[_full((G, 1)), _full((G, EMB))],
        out_shape=[_sds((G, 1)), _sds((G, EMB))],
        scratch_shapes=[pltpu.VMEM((G, 32), f32), pltpu.VMEM((G, 1), f32)],
    )(a3a, a3b, b3r, batr, A1w, A1b.reshape(1, -1), A2w, A2b.reshape(1, -1),
      A3w, A3b.reshape(1, -1), GEw, GEb.reshape(1, -1))
    return (anomaly, emb)


# speedup vs baseline: 1.0221x; 1.0066x over previous
import functools

import jax
import jax.numpy as jnp
from jax import lax
from jax.experimental import pallas as pl
from jax.experimental.pallas import tpu as pltpu
from jax.experimental.pallas import tpu_sc as plsc

N = 50000
E = 800000
F_IN = 32
HID = 64
HEADS = 4
G = 64
EMB = 128

NC = 2
NS = 16
NWK = NC * NS
L = 16

NP = 50176
RPT = NP // NS

EDGES = E + N
W = 512
NWIN = 52
EP = NWK * W * NWIN
EPT = EP // NWK
EPA = EP + 2 * W

NB = 1024
NBLK = NP // NB

_sc_params = pltpu.CompilerParams(use_tc_tiling_on_sc=False)


def _mesh():
    return plsc.VectorSubcoreMesh(
        core_axis_name="c", subcore_axis_name="s",
        num_cores=NC, num_subcores=NS)


def _zero_shared(z_h, acc, sid):
    pltpu.sync_copy(z_h.at[pl.ds(sid * RPT, RPT)],
                    acc.at[pl.ds(sid * RPT, RPT)])


def _dump_shared(acc, out, c, sid):
    @pl.when(c == 0)
    def _c0():
        pltpu.sync_copy(acc.at[pl.ds(sid * RPT, RPT)],
                        out.at[0, pl.ds(sid * RPT, RPT)])

    @pl.when(c == 1)
    def _c1():
        pltpu.sync_copy(acc.at[pl.ds(sid * RPT, RPT)],
                        out.at[1, pl.ds(sid * RPT, RPT)])


def _make_edge_kernel(gat):
    out_type = [jax.ShapeDtypeStruct((EPA, L), jnp.float32)]
    if gat:
        out_type.append(jax.ShapeDtypeStruct((NC, NP, L), jnp.float32))
    scratch = [
        pltpu.VMEM((W,), jnp.int32), pltpu.VMEM((W,), jnp.int32),
        pltpu.VMEM((W,), jnp.int32), pltpu.VMEM((W,), jnp.int32),
        pltpu.VMEM((W, L), jnp.float32), pltpu.VMEM((W, L), jnp.float32),
        pltpu.VMEM((W, L), jnp.float32), pltpu.VMEM((W, L), jnp.float32),
        pltpu.SemaphoreType.DMA, pltpu.SemaphoreType.DMA,
        pltpu.SemaphoreType.DMA, pltpu.SemaphoreType.DMA,
        pltpu.SemaphoreType.DMA, pltpu.SemaphoreType.DMA,
    ]
    if gat:
        scratch.append(pltpu.VMEM_SHARED((NP, L), jnp.float32))

    @functools.partial(pl.kernel, out_type=tuple(out_type), mesh=_mesh(),
                       scratch_types=scratch, compiler_params=_sc_params)
    def edge_kernel(s_tab, d_tab, src_h, dst_h, z_h, *rest):
        it = iter(rest)
        ex_o = next(it)
        den_o = next(it) if gat else None
        srcb = [next(it), next(it)]
        dstb = [next(it), next(it)]
        sr = [next(it), next(it)]
        dr = [next(it), next(it)]
        sems = [next(it), next(it)], [next(it), next(it)]
        sem_i = [next(it), next(it)]
        den_acc = next(it) if gat else None

        c = lax.axis_index("c")
        sid = lax.axis_index("s")
        wkr = sid * NC + c
        if gat:
            _zero_shared(z_h, den_acc, sid)
        plsc.subcore_barrier()

        def issue_idx(b, w):
            base = wkr * EPT + w * W
            i1 = pltpu.async_copy(src_h.at[pl.ds(base, W)], srcb[b],
                                  sem_i[b])
            i2 = pltpu.async_copy(dst_h.at[pl.ds(base, W)], dstb[b],
                                  sem_i[b])
            return (i1, i2)

        def issue(b, idx_descs):
            for d in idx_descs:
                d.wait()
            g1 = pltpu.async_copy(s_tab.at[srcb[b]], sr[b], sems[0][b])
            g2 = pltpu.async_copy(d_tab.at[dstb[b]], dr[b], sems[1][b])
            return (g1, g2)

        def process(b, w, descs):
            base = wkr * EPT + w * W
            for d in descs:
                d.wait()

            def ebody(j, cc):
                if gat:
                    e = sr[b][j] + dr[b][j]
                    e = jnp.where(e >= 0.0, e, 0.2 * e)
                    sr[b][j] = jnp.exp(e)
                else:
                    sr[b][j] = sr[b][j] * dr[b][j]
                return cc

            lax.fori_loop(0, W, ebody, None, unroll=8)
            pltpu.sync_copy(sr[b], ex_o.at[pl.ds(base, W)])
            if gat:
                pltpu.sync_copy(sr[b], den_acc.at[dstb[b]], add=True)

        def pair(t, carry):
            w0 = 2 * t
            i0 = issue_idx(0, w0)
            i1 = issue_idx(1, w0 + 1)
            d0 = issue(0, i0)
            d1 = issue(1, i1)
            process(0, w0, d0)
            process(1, w0 + 1, d1)
            return carry

        lax.fori_loop(0, NWIN // 2, pair, None)
        if gat:
            plsc.subcore_barrier()
            _dump_shared(den_acc, den_o, c, sid)

    return edge_kernel


def _make_chunk_kernel(h):
    @functools.partial(
        pl.kernel,
        out_type=jax.ShapeDtypeStruct((NC, NP, L), jnp.float32),
        mesh=_mesh(),
        scratch_types=[
            pltpu.VMEM((W,), jnp.int32), pltpu.VMEM((W,), jnp.int32),
            pltpu.VMEM((W,), jnp.int32), pltpu.VMEM((W,), jnp.int32),
            pltpu.VMEM((W, L), jnp.float32), pltpu.VMEM((W, L), jnp.float32),
            pltpu.VMEM((W, L), jnp.float32), pltpu.VMEM((W, L), jnp.float32),
            pltpu.VMEM_SHARED((NP, L), jnp.float32),
            pltpu.SemaphoreType.DMA, pltpu.SemaphoreType.DMA,
            pltpu.SemaphoreType.DMA, pltpu.SemaphoreType.DMA,
            pltpu.SemaphoreType.DMA, pltpu.SemaphoreType.DMA,
        ],
        compiler_params=_sc_params)
    def chunk_kernel(ta, exr, src_h, dst_h, z_h, out, *bufs):
        it = iter(bufs)
        srcb = [next(it), next(it)]
        dstb = [next(it), next(it)]
        exb = [next(it), next(it)]
        rows_a = [next(it), next(it)]
        acc_a = next(it)
        sem_e = [next(it), next(it)]
        sem_a = [next(it), next(it)]
        sem_i = [next(it), next(it)]

        c = lax.axis_index("c")
        sid = lax.axis_index("s")
        wkr = sid * NC + c
        _zero_shared(z_h, acc_a, sid)
        plsc.subcore_barrier()

        def issue_idx(b, w):
            base = wkr * EPT + w * W
            i1 = pltpu.async_copy(src_h.at[pl.ds(base, W)], srcb[b],
                                  sem_i[b])
            i2 = pltpu.async_copy(dst_h.at[pl.ds(base, W)], dstb[b],
                                  sem_i[b])
            return (i1, i2)

        def issue(b, w, idx_descs):
            base = wkr * EPT + w * W
            g1 = pltpu.async_copy(exr.at[pl.ds(base, W)], exb[b], sem_e[b])
            for d in idx_descs:
                d.wait()
            g2 = pltpu.async_copy(ta.at[srcb[b]], rows_a[b], sem_a[b])
            return (g1, g2)

        def process(b, descs):
            for d in descs:
                d.wait()

            def ebody(j, cc):
                ev = jnp.broadcast_to(exb[b][j][h:h + 1], (L,))
                rows_a[b][j] = rows_a[b][j] * ev
                return cc

            lax.fori_loop(0, W, ebody, None, unroll=8)
            pltpu.sync_copy(rows_a[b], acc_a.at[dstb[b]], add=True)

        def pair(t, carry):
            w0 = 2 * t
            i0 = issue_idx(0, w0)
            i1 = issue_idx(1, w0 + 1)
            d0 = issue(0, w0, i0)
            d1 = issue(1, w0 + 1, i1)
            process(0, d0)
            process(1, d1)
            return carry

        lax.fori_loop(0, NWIN // 2, pair, None)
        plsc.subcore_barrier()
        _dump_shared(acc_a, out, c, sid)

    return chunk_kernel


@functools.cache
def _sc_kernels():
    return {
        "gat": _make_edge_kernel(gat=True),
        "gcn": _make_edge_kernel(gat=False),
        "chunk": [_make_chunk_kernel(hh) for hh in range(4)],
    }


def _prep1_body(xp, w1, as1, ad1, *outs):
    xw_refs = outs[:16]
    s_o, d_o = outs[16], outs[17]
    xw = jnp.dot(xp[...], w1[...], preferred_element_type=jnp.float32)
    for t in range(16):
        xw_refs[t][...] = xw[:, t * L:(t + 1) * L]
    z12 = jnp.zeros((NB, L - HEADS), jnp.float32)
    s_cols = []
    d_cols = []
    for hh in range(HEADS):
        blk = xw[:, hh * HID:(hh + 1) * HID]
        s_cols.append(jnp.sum(blk * as1[...][hh][None, :], axis=1,
                              keepdims=True))
        d_cols.append(jnp.sum(blk * ad1[...][hh][None, :], axis=1,
                              keepdims=True))
    s_o[...] = jnp.concatenate(s_cols + [z12], axis=1)
    d_o[...] = jnp.concatenate(d_cols + [z12], axis=1)


def _prep2_body(*args):
    accs = args[:16]
    den, b1r, w2, as2, ad2 = args[16:21]
    outs = args[21:]
    xw2_refs = outs[:4]
    s2_o, d2_o, dinv_o = outs[4], outs[5], outs[6]
    dn = den[0] + den[1]
    parts = []
    for t in range(16):
        numt = accs[t][0] + accs[t][1]
        hh = t // 4
        ht = numt / (dn[:, hh:hh + 1] + 1e-16) \
            + b1r[...][:, t * L:(t + 1) * L]
        parts.append(jnp.maximum(ht, 0.0))
    h1 = jnp.concatenate(parts, axis=1)
    xw2 = jnp.dot(h1, w2[...], preferred_element_type=jnp.float32)
    for t in range(4):
        xw2_refs[t][...] = xw2[:, t * L:(t + 1) * L]
    z15 = jnp.zeros((NB, L - 1), jnp.float32)
    s2_o[...] = jnp.concatenate(
        [jnp.sum(xw2 * as2[...], axis=1, keepdims=True), z15], axis=1)
    d2_o[...] = jnp.concatenate(
        [jnp.sum(xw2 * ad2[...], axis=1, keepdims=True), z15], axis=1)
    deg = dn[:, HEADS:HEADS + 1]
    dinv = jnp.where(deg > 0.0, lax.rsqrt(jnp.maximum(deg, 1e-30)), 0.0)
    dinv_o[...] = jnp.concatenate([dinv, z15], axis=1)


def _prep3_body(a0, a1, a2, a3, den2, b2r, w3, x3a, x3b):
    num = jnp.concatenate([a0[0] + a0[1], a1[0] + a1[1], a2[0] + a2[1],
                           a3[0] + a3[1]], axis=1)
    dn = den2[0][:, 0:1] + den2[1][:, 0:1]
    h2 = jnp.maximum(num / (dn + 1e-16) + b2r[...], 0.0)
    xw3 = jnp.dot(h2, w3[...], preferred_element_type=jnp.float32)
    x3a[...] = xw3[:, :L]
    x3b[...] = xw3[:, L:]


def _final_body(a3a, a3b, b3r, batr, a1w, a1b, a2w, a2b, a3w, a3b_, gew, geb,
                an_o, emb_o, gm_acc, cnt_acc):
    i = pl.program_id(0)

    @pl.when(i == 0)
    def _init():
        gm_acc[...] = jnp.zeros_like(gm_acc)
        cnt_acc[...] = jnp.zeros_like(cnt_acc)

    h3 = jnp.concatenate([a3a[0] + a3a[1], a3b[0] + a3b[1]], axis=1)
    h3 = jnp.maximum(h3 + b3r[...], 0.0)
    bt = batr[...]
    gi = lax.broadcasted_iota(jnp.int32, (G, NB), 0).astype(jnp.float32)
    mask = (bt == gi).astype(jnp.float32)
    gm_acc[...] = gm_acc[...] + jnp.dot(mask, h3,
                                        preferred_element_type=jnp.float32)
    cnt_acc[...] = cnt_acc[...] + jnp.sum(mask, axis=1, keepdims=True)

    @pl.when(i == NBLK - 1)
    def _fin():
        gm = gm_acc[...] / jnp.maximum(cnt_acc[...], 1.0)
        a = jnp.maximum(jnp.dot(gm, a1w[...]) + a1b[...], 0.0)
        a = jnp.maximum(jnp.dot(a, a2w[...]) + a2b[...], 0.0)
        an_o[...] = jax.nn.sigmoid(jnp.dot(a, a3w[...]) + a3b_[...])
        emb_o[...] = jnp.tanh(jnp.dot(gm, gew[...]) + geb[...])


def _full(shape):
    return pl.BlockSpec(shape, lambda i: tuple(0 for _ in shape))


def _nblk(cols):
    return pl.BlockSpec((NB, cols), lambda i: (i, 0))


def _accblk(cols):
    return pl.BlockSpec((NC, NB, cols), lambda i: (0, i, 0))


def _sds(shape):
    return jax.ShapeDtypeStruct(shape, jnp.float32)


def kernel(x, edge_index, batch, W1, a_src1, a_dst1, b1, W2, a_src2, a_dst2,
           b2, W3, b3, A1w, A1b, A2w, A2b, A3w, A3b, GEw, GEb):
    f32 = jnp.float32
    xp = jnp.pad(x, ((0, NP - N), (0, 0)))
    loop = jnp.arange(N, dtype=jnp.int32)
    padi = N + (jnp.arange(EPA - EDGES, dtype=jnp.int32) % (NP - N))
    src = jnp.concatenate([edge_index[0], loop, padi])
    dst = jnp.concatenate([edge_index[1], loop, padi])
    batr = jnp.pad(batch, (0, NP - N), constant_values=G).astype(f32)
    batr = batr.reshape(1, NP)
    b1r = b1.reshape(1, -1)
    b2r = b2.reshape(1, -1)
    b3r = b3.reshape(1, -1)
    zeros16 = jnp.zeros((NP, L), f32)

    p1 = pl.pallas_call(
        _prep1_body,
        grid=(NBLK,),
        in_specs=[_nblk(F_IN), _full((F_IN, HEADS * HID)),
                  _full((HEADS, HID)), _full((HEADS, HID))],
        out_specs=[_nblk(L)] * 18,
        out_shape=[_sds((NP, L))] * 18,
    )(xp, W1, a_src1, a_dst1)
    xw1 = p1[:16]
    s1, d1 = p1[16], p1[17]

    sck = _sc_kernels()
    ex1, den1 = sck["gat"](s1, d1, src, dst, zeros16)
    acc1 = [sck["chunk"][t // 4](xw1[t], ex1, src, dst, zeros16)
            for t in range(16)]

    p2 = pl.pallas_call(
        _prep2_body,
        grid=(NBLK,),
        in_specs=[_accblk(L)] * 17
        + [_full((1, HEADS * HID)), _full((HEADS * HID, HID)),
           _full((1, HID)), _full((1, HID))],
        out_specs=[_nblk(L)] * 7,
        out_shape=[_sds((NP, L))] * 7,
    )(*acc1, den1, b1r, W2, a_src2, a_dst2)
    xw2 = p2[:4]
    s2, d2, dinv = p2[4], p2[5], p2[6]

    ex2, den2 = sck["gat"](s2, d2, src, dst, zeros16)
    a2_00 = sck["chunk"][0](xw2[0], ex2, src, dst, zeros16)
    a2_01 = sck["chunk"][0](xw2[1], ex2, src, dst, zeros16)
    a2_10 = sck["chunk"][0](xw2[2], ex2, src, dst, zeros16)
    a2_11 = sck["chunk"][0](xw2[3], ex2, src, dst, zeros16)

    x3a, x3b = pl.pallas_call(
        _prep3_body,
        grid=(NBLK,),
        in_specs=[_accblk(L)] * 5 + [_full((1, HID)),
                                     _full((HID, HID // 2))],
        out_specs=[_nblk(L), _nblk(L)],
        out_shape=[_sds((NP, L)), _sds((NP, L))],
    )(a2_00, a2_01, a2_10, a2_11, den2, b2r, W3)

    (norm,) = sck["gcn"](dinv, dinv, src, dst, zeros16)
    a3a = sck["chunk"][0](x3a, norm, src, dst, zeros16)
    a3b = sck["chunk"][0](x3b, norm, src, dst, zeros16)

    anomaly, emb = pl.pallas_call(
        _final_body,
        grid=(NBLK,),
        in_specs=[_accblk(L), _accblk(L), _full((1, 32)),
                  pl.BlockSpec((1, NB), lambda i: (0, i)),
                  _full((32, 32)), _full((1, 32)), _full((32, 16)),
                  _full((1, 16)), _full((16, 1)), _full((1, 1)),
                  _full((32, EMB)), _full((1, EMB))],
        out_specs=[_full((G, 1)), _full((G, EMB))],
        out_shape=[_sds((G, 1)), _sds((G, EMB))],
        scratch_shapes=[pltpu.VMEM((G, 32), f32), pltpu.VMEM((G, 1), f32)],
    )(a3a, a3b, b3r, batr, A1w, A1b.reshape(1, -1), A2w, A2b.reshape(1, -1),
      A3w, A3b.reshape(1, -1), GEw, GEb.reshape(1, -1))
    return (anomaly, emb)


# speedup vs baseline: 1.0505x; 1.0278x over previous
import functools

import jax
import jax.numpy as jnp
from jax import lax
from jax.experimental import pallas as pl
from jax.experimental.pallas import tpu as pltpu
from jax.experimental.pallas import tpu_sc as plsc

N = 50000
E = 800000
F_IN = 32
HID = 64
HEADS = 4
G = 64
EMB = 128

NC = 2
NS = 16
NWK = NC * NS
L = 16

NP = 50176
RPT = NP // NS

EDGES = E + N
W = 512
NWIN = 52
EP = NWK * W * NWIN
EPT = EP // NWK
EPA = EP + 2 * W

NB = 1024
NBLK = NP // NB

_sc_params = pltpu.CompilerParams(use_tc_tiling_on_sc=False)


def _mesh():
    return plsc.VectorSubcoreMesh(
        core_axis_name="c", subcore_axis_name="s",
        num_cores=NC, num_subcores=NS)


def _zero_shared(z_h, acc, sid):
    pltpu.sync_copy(z_h.at[pl.ds(sid * RPT, RPT)],
                    acc.at[pl.ds(sid * RPT, RPT)])


def _dump_shared(acc, out, c, sid):
    @pl.when(c == 0)
    def _c0():
        pltpu.sync_copy(acc.at[pl.ds(sid * RPT, RPT)],
                        out.at[0, pl.ds(sid * RPT, RPT)])

    @pl.when(c == 1)
    def _c1():
        pltpu.sync_copy(acc.at[pl.ds(sid * RPT, RPT)],
                        out.at[1, pl.ds(sid * RPT, RPT)])


def _make_edge_kernel(gat):
    out_type = [jax.ShapeDtypeStruct((EPA, L), jnp.float32)]
    if gat:
        out_type.append(jax.ShapeDtypeStruct((NC, NP, L), jnp.float32))
    scratch = [
        pltpu.VMEM((W,), jnp.int32), pltpu.VMEM((W,), jnp.int32),
        pltpu.VMEM((W,), jnp.int32), pltpu.VMEM((W,), jnp.int32),
        pltpu.VMEM((W, L), jnp.float32), pltpu.VMEM((W, L), jnp.float32),
        pltpu.VMEM((W, L), jnp.float32), pltpu.VMEM((W, L), jnp.float32),
        pltpu.SemaphoreType.DMA, pltpu.SemaphoreType.DMA,
        pltpu.SemaphoreType.DMA, pltpu.SemaphoreType.DMA,
        pltpu.SemaphoreType.DMA, pltpu.SemaphoreType.DMA,
    ]
    if gat:
        scratch.append(pltpu.VMEM_SHARED((NP, L), jnp.float32))

    @functools.partial(pl.kernel, out_type=tuple(out_type), mesh=_mesh(),
                       scratch_types=scratch, compiler_params=_sc_params)
    def edge_kernel(s_tab, d_tab, src_h, dst_h, z_h, *rest):
        it = iter(rest)
        ex_o = next(it)
        den_o = next(it) if gat else None
        srcb = [next(it), next(it)]
        dstb = [next(it), next(it)]
        sr = [next(it), next(it)]
        dr = [next(it), next(it)]
        sems = [next(it), next(it)], [next(it), next(it)]
        sem_i = [next(it), next(it)]
        den_acc = next(it) if gat else None

        c = lax.axis_index("c")
        sid = lax.axis_index("s")
        wkr = sid * NC + c
        if gat:
            _zero_shared(z_h, den_acc, sid)
        plsc.subcore_barrier()

        def issue_idx(b, w):
            base = wkr * EPT + w * W
            i1 = pltpu.async_copy(src_h.at[pl.ds(base, W)], srcb[b],
                                  sem_i[b])
            i2 = pltpu.async_copy(dst_h.at[pl.ds(base, W)], dstb[b],
                                  sem_i[b])
            return (i1, i2)

        def issue(b, idx_descs):
            for d in idx_descs:
                d.wait()
            g1 = pltpu.async_copy(s_tab.at[srcb[b]], sr[b], sems[0][b])
            g2 = pltpu.async_copy(d_tab.at[dstb[b]], dr[b], sems[1][b])
            return (g1, g2)

        def process(b, w, descs):
            base = wkr * EPT + w * W
            for d in descs:
                d.wait()

            def ebody(j, cc):
                if gat:
                    e = sr[b][j] + dr[b][j]
                    e = jnp.where(e >= 0.0, e, 0.2 * e)
                    sr[b][j] = jnp.exp(e)
                else:
                    sr[b][j] = sr[b][j] * dr[b][j]
                return cc

            lax.fori_loop(0, W, ebody, None, unroll=8)
            pltpu.sync_copy(sr[b], ex_o.at[pl.ds(base, W)])
            if gat:
                pltpu.sync_copy(sr[b], den_acc.at[dstb[b]], add=True)

        def pair(t, carry):
            w0 = 2 * t
            i0 = issue_idx(0, w0)
            i1 = issue_idx(1, w0 + 1)
            d0 = issue(0, i0)
            d1 = issue(1, i1)
            process(0, w0, d0)
            process(1, w0 + 1, d1)
            return carry

        lax.fori_loop(0, NWIN // 2, pair, None)
        if gat:
            plsc.subcore_barrier()
            _dump_shared(den_acc, den_o, c, sid)

    return edge_kernel


def _make_chunk_kernel(h):
    @functools.partial(
        pl.kernel,
        out_type=jax.ShapeDtypeStruct((NC, NP, L), jnp.float32),
        mesh=_mesh(),
        scratch_types=[
            pltpu.VMEM((W,), jnp.int32), pltpu.VMEM((W,), jnp.int32),
            pltpu.VMEM((W,), jnp.int32), pltpu.VMEM((W,), jnp.int32),
            pltpu.VMEM((W, L), jnp.float32), pltpu.VMEM((W, L), jnp.float32),
            pltpu.VMEM((W, L), jnp.float32), pltpu.VMEM((W, L), jnp.float32),
            pltpu.VMEM_SHARED((NP, L), jnp.float32),
            pltpu.SemaphoreType.DMA, pltpu.SemaphoreType.DMA,
            pltpu.SemaphoreType.DMA, pltpu.SemaphoreType.DMA,
            pltpu.SemaphoreType.DMA, pltpu.SemaphoreType.DMA,
            pltpu.SemaphoreType.DMA, pltpu.SemaphoreType.DMA,
        ],
        compiler_params=_sc_params)
    def chunk_kernel(ta, exr, src_h, dst_h, z_h, out, *bufs):
        it = iter(bufs)
        srcb = [next(it), next(it)]
        dstb = [next(it), next(it)]
        exb = [next(it), next(it)]
        rows_a = [next(it), next(it)]
        acc_a = next(it)
        sem_e = [next(it), next(it)]
        sem_a = [next(it), next(it)]
        sem_i = [next(it), next(it)]
        sem_s = [next(it), next(it)]

        c = lax.axis_index("c")
        sid = lax.axis_index("s")
        wkr = sid * NC + c
        _zero_shared(z_h, acc_a, sid)
        plsc.subcore_barrier()

        def issue_idx(b, w):
            base = wkr * EPT + w * W
            i1 = pltpu.async_copy(src_h.at[pl.ds(base, W)], srcb[b],
                                  sem_i[b])
            i2 = pltpu.async_copy(dst_h.at[pl.ds(base, W)], dstb[b],
                                  sem_i[b])
            return (i1, i2)

        def issue(b, w, idx_descs):
            base = wkr * EPT + w * W
            g1 = pltpu.async_copy(exr.at[pl.ds(base, W)], exb[b], sem_e[b])
            for d in idx_descs:
                d.wait()
            g2 = pltpu.async_copy(ta.at[srcb[b]], rows_a[b], sem_a[b])
            return (g1, g2)

        def compute(b, descs):
            for d in descs:
                d.wait()

            def ebody(j, cc):
                ev = jnp.broadcast_to(exb[b][j][h:h + 1], (L,))
                rows_a[b][j] = rows_a[b][j] * ev
                return cc

            lax.fori_loop(0, W, ebody, None, unroll=8)
            return pltpu.async_copy(rows_a[b], acc_a.at[dstb[b]],
                                    sem_s[b], add=True)

        def pair(t, carry):
            w0 = 2 * t
            i0 = issue_idx(0, w0)
            i1 = issue_idx(1, w0 + 1)
            d0 = issue(0, w0, i0)
            d1 = issue(1, w0 + 1, i1)
            s0 = compute(0, d0)
            s1 = compute(1, d1)
            s0.wait()
            s1.wait()
            return carry

        lax.fori_loop(0, NWIN // 2, pair, None)
        plsc.subcore_barrier()
        _dump_shared(acc_a, out, c, sid)

    return chunk_kernel


@functools.cache
def _sc_kernels():
    return {
        "gat": _make_edge_kernel(gat=True),
        "gcn": _make_edge_kernel(gat=False),
        "chunk": [_make_chunk_kernel(hh) for hh in range(4)],
    }


def _prep1_body(xp, w1, as1, ad1, *outs):
    xw_refs = outs[:16]
    s_o, d_o = outs[16], outs[17]
    xw = jnp.dot(xp[...], w1[...], preferred_element_type=jnp.float32)
    for t in range(16):
        xw_refs[t][...] = xw[:, t * L:(t + 1) * L]
    z12 = jnp.zeros((NB, L - HEADS), jnp.float32)
    s_cols = []
    d_cols = []
    for hh in range(HEADS):
        blk = xw[:, hh * HID:(hh + 1) * HID]
        s_cols.append(jnp.sum(blk * as1[...][hh][None, :], axis=1,
                              keepdims=True))
        d_cols.append(jnp.sum(blk * ad1[...][hh][None, :], axis=1,
                              keepdims=True))
    s_o[...] = jnp.concatenate(s_cols + [z12], axis=1)
    d_o[...] = jnp.concatenate(d_cols + [z12], axis=1)


def _prep2_body(*args):
    accs = args[:16]
    den, b1r, w2, as2, ad2 = args[16:21]
    outs = args[21:]
    xw2_refs = outs[:4]
    s2_o, d2_o, dinv_o = outs[4], outs[5], outs[6]
    dn = den[0] + den[1]
    parts = []
    for t in range(16):
        numt = accs[t][0] + accs[t][1]
        hh = t // 4
        ht = numt / (dn[:, hh:hh + 1] + 1e-16) \
            + b1r[...][:, t * L:(t + 1) * L]
        parts.append(jnp.maximum(ht, 0.0))
    h1 = jnp.concatenate(parts, axis=1)
    xw2 = jnp.dot(h1, w2[...], preferred_element_type=jnp.float32)
    for t in range(4):
        xw2_refs[t][...] = xw2[:, t * L:(t + 1) * L]
    z15 = jnp.zeros((NB, L - 1), jnp.float32)
    s2_o[...] = jnp.concatenate(
        [jnp.sum(xw2 * as2[...], axis=1, keepdims=True), z15], axis=1)
    d2_o[...] = jnp.concatenate(
        [jnp.sum(xw2 * ad2[...], axis=1, keepdims=True), z15], axis=1)
    deg = dn[:, HEADS:HEADS + 1]
    dinv = jnp.where(deg > 0.0, lax.rsqrt(jnp.maximum(deg, 1e-30)), 0.0)
    dinv_o[...] = jnp.concatenate([dinv, z15], axis=1)


def _prep3_body(a0, a1, a2, a3, den2, b2r, w3, x3a, x3b):
    num = jnp.concatenate([a0[0] + a0[1], a1[0] + a1[1], a2[0] + a2[1],
                           a3[0] + a3[1]], axis=1)
    dn = den2[0][:, 0:1] + den2[1][:, 0:1]
    h2 = jnp.maximum(num / (dn + 1e-16) + b2r[...], 0.0)
    xw3 = jnp.dot(h2, w3[...], preferred_element_type=jnp.float32)
    x3a[...] = xw3[:, :L]
    x3b[...] = xw3[:, L:]


def _final_body(a3a, a3b, b3r, batr, a1w, a1b, a2w, a2b, a3w, a3b_, gew, geb,
                an_o, emb_o, gm_acc, cnt_acc):
    i = pl.program_id(0)

    @pl.when(i == 0)
    def _init():
        gm_acc[...] = jnp.zeros_like(gm_acc)
        cnt_acc[...] = jnp.zeros_like(cnt_acc)

    h3 = jnp.concatenate([a3a[0] + a3a[1], a3b[0] + a3b[1]], axis=1)
    h3 = jnp.maximum(h3 + b3r[...], 0.0)
    bt = batr[...]
    gi = lax.broadcasted_iota(jnp.int32, (G, NB), 0).astype(jnp.float32)
    mask = (bt == gi).astype(jnp.float32)
    gm_acc[...] = gm_acc[...] + jnp.dot(mask, h3,
                                        preferred_element_type=jnp.float32)
    cnt_acc[...] = cnt_acc[...] + jnp.sum(mask, axis=1, keepdims=True)

    @pl.when(i == NBLK - 1)
    def _fin():
        gm = gm_acc[...] / jnp.maximum(cnt_acc[...], 1.0)
        a = jnp.maximum(jnp.dot(gm, a1w[...]) + a1b[...], 0.0)
        a = jnp.maximum(jnp.dot(a, a2w[...]) + a2b[...], 0.0)
        an_o[...] = jax.nn.sigmoid(jnp.dot(a, a3w[...]) + a3b_[...])
        emb_o[...] = jnp.tanh(jnp.dot(gm, gew[...]) + geb[...])


def _full(shape):
    return pl.BlockSpec(shape, lambda i: tuple(0 for _ in shape))


def _nblk(cols):
    return pl.BlockSpec((NB, cols), lambda i: (i, 0))


def _accblk(cols):
    return pl.BlockSpec((NC, NB, cols), lambda i: (0, i, 0))


def _sds(shape):
    return jax.ShapeDtypeStruct(shape, jnp.float32)


def kernel(x, edge_index, batch, W1, a_src1, a_dst1, b1, W2, a_src2, a_dst2,
           b2, W3, b3, A1w, A1b, A2w, A2b, A3w, A3b, GEw, GEb):
    f32 = jnp.float32
    xp = jnp.pad(x, ((0, NP - N), (0, 0)))
    loop = jnp.arange(N, dtype=jnp.int32)
    padi = N + (jnp.arange(EPA - EDGES, dtype=jnp.int32) % (NP - N))
    src = jnp.concatenate([edge_index[0], loop, padi])
    dst = jnp.concatenate([edge_index[1], loop, padi])
    batr = jnp.pad(batch, (0, NP - N), constant_values=G).astype(f32)
    batr = batr.reshape(1, NP)
    b1r = b1.reshape(1, -1)
    b2r = b2.reshape(1, -1)
    b3r = b3.reshape(1, -1)
    zeros16 = jnp.zeros((NP, L), f32)

    p1 = pl.pallas_call(
        _prep1_body,
        grid=(NBLK,),
        in_specs=[_nblk(F_IN), _full((F_IN, HEADS * HID)),
                  _full((HEADS, HID)), _full((HEADS, HID))],
        out_specs=[_nblk(L)] * 18,
        out_shape=[_sds((NP, L))] * 18,
    )(xp, W1, a_src1, a_dst1)
    xw1 = p1[:16]
    s1, d1 = p1[16], p1[17]

    sck = _sc_kernels()
    ex1, den1 = sck["gat"](s1, d1, src, dst, zeros16)
    acc1 = [sck["chunk"][t // 4](xw1[t], ex1, src, dst, zeros16)
            for t in range(16)]

    p2 = pl.pallas_call(
        _prep2_body,
        grid=(NBLK,),
        in_specs=[_accblk(L)] * 17
        + [_full((1, HEADS * HID)), _full((HEADS * HID, HID)),
           _full((1, HID)), _full((1, HID))],
        out_specs=[_nblk(L)] * 7,
        out_shape=[_sds((NP, L))] * 7,
    )(*acc1, den1, b1r, W2, a_src2, a_dst2)
    xw2 = p2[:4]
    s2, d2, dinv = p2[4], p2[5], p2[6]

    ex2, den2 = sck["gat"](s2, d2, src, dst, zeros16)
    a2_00 = sck["chunk"][0](xw2[0], ex2, src, dst, zeros16)
    a2_01 = sck["chunk"][0](xw2[1], ex2, src, dst, zeros16)
    a2_10 = sck["chunk"][0](xw2[2], ex2, src, dst, zeros16)
    a2_11 = sck["chunk"][0](xw2[3], ex2, src, dst, zeros16)

    x3a, x3b = pl.pallas_call(
        _prep3_body,
        grid=(NBLK,),
        in_specs=[_accblk(L)] * 5 + [_full((1, HID)),
                                     _full((HID, HID // 2))],
        out_specs=[_nblk(L), _nblk(L)],
        out_shape=[_sds((NP, L)), _sds((NP, L))],
    )(a2_00, a2_01, a2_10, a2_11, den2, b2r, W3)

    (norm,) = sck["gcn"](dinv, dinv, src, dst, zeros16)
    a3a = sck["chunk"][0](x3a, norm, src, dst, zeros16)
    a3b = sck["chunk"][0](x3b, norm, src, dst, zeros16)

    anomaly, emb = pl.pallas_call(
        _final_body,
        grid=(NBLK,),
        in_specs=[_accblk(L), _accblk(L), _full((1, 32)),
                  pl.BlockSpec((1, NB), lambda i: (0, i)),
                  _full((32, 32)), _full((1, 32)), _full((32, 16)),
                  _full((1, 16)), _full((16, 1)), _full((1, 1)),
                  _full((32, EMB)), _full((1, EMB))],
        out_specs=[_full((G, 1)), _full((G, EMB))],
        out_shape=[_sds((G, 1)), _sds((G, EMB))],
        scratch_shapes=[pltpu.VMEM((G, 32), f32), pltpu.VMEM((G, 1), f32)],
    )(a3a, a3b, b3r, batr, A1w, A1b.reshape(1, -1), A2w, A2b.reshape(1, -1),
      A3w, A3b.reshape(1, -1), GEw, GEb.reshape(1, -1))
    return (anomaly, emb)
